# baseline (speedup 1.0000x reference)
import functools

import jax
import jax.numpy as jnp
from jax import lax
from jax.experimental import pallas as pl
from jax.experimental.pallas import tpu as pltpu
from jax.experimental.pallas import tpu_sc as plsc

K = 16
N = 2048
BZ = 4
D = 128
RA = 256
RC = 128

NC = 2
NS = 16
NW = NC * NS
TOT_ROWS = BZ * N * K
ROWS_PER_W = TOT_ROWS // NW
CHUNK = 128
NCHUNK = ROWS_PER_W // CHUNK


def _knn_body(pos_ref, post_ref, nn_ref):
    b = pl.program_id(0)
    pq = pos_ref[0]
    pk = post_ref[0]
    dist = None
    for d in range(3):
        dd = pq[:, d:d + 1] - pk[d:d + 1, :]
        sq = dd * dd
        dist = sq if dist is None else dist + sq
    iota = lax.broadcasted_iota(jnp.int32, (RA, N), 1).astype(jnp.float32)
    kio = lax.broadcasted_iota(jnp.int32, (RA, K), 1)
    out = jnp.zeros((RA, K), jnp.float32)
    inf = jnp.float32(jnp.inf)
    for k in range(K):
        m = jnp.min(dist, axis=1, keepdims=True)
        cand = jnp.where(dist == m, iota, float(N))
        idx = jnp.min(cand, axis=1, keepdims=True)
        out = jnp.where(kio == k, idx, out)
        dist = jnp.where(cand == idx, inf, dist)
    nn_ref[0] = out.astype(jnp.int32) + b * N


def _knn(pos, post):
    bz = pos.shape[0]
    return pl.pallas_call(
        _knn_body,
        grid=(bz, N // RA),
        in_specs=[
            pl.BlockSpec((1, RA, 3), lambda b, i: (b, i, 0)),
            pl.BlockSpec((1, 3, N), lambda b, i: (b, 0, 0)),
        ],
        out_specs=pl.BlockSpec((1, RA, K), lambda b, i: (b, i, 0)),
        out_shape=jax.ShapeDtypeStruct((bz, N, K), jnp.int32),
    )(pos, post)


def _gather_body(rows_pw, nchunk, table_hbm, idx_hbm, out_hbm, idx_v,
                 rows0, rows1, g0, g1, s0, s1):
    NCHUNK = nchunk
    wid = lax.axis_index("s") * NC + lax.axis_index("c")
    wbase = pl.multiple_of(wid * rows_pw, rows_pw)
    pltpu.sync_copy(idx_hbm.at[pl.ds(wbase, rows_pw)], idx_v)

    rows = (rows0, rows1)
    gsem = (g0, g1)
    ssem = (s0, s1)

    def gather_start(i):
        buf = i % 2
        return pltpu.async_copy(
            table_hbm.at[idx_v.at[pl.ds(i * CHUNK, CHUNK)]],
            rows[buf], gsem[buf])

    def scatter_start(i):
        buf = i % 2
        base = pl.multiple_of(wbase + i * CHUNK, CHUNK)
        return pltpu.async_copy(rows[buf], out_hbm.at[pl.ds(base, CHUNK)],
                                ssem[buf])

    pending = {0: gather_start(0)}
    scat = {}
    for i in range(NCHUNK):
        buf = i % 2
        if i + 1 < NCHUNK:
            if i >= 1:
                scat[i - 1].wait()
            pending[i + 1] = gather_start(i + 1)
        pending[i].wait()
        scat[i] = scatter_start(i)
    scat[NCHUNK - 2].wait()
    scat[NCHUNK - 1].wait()


def _sc_gather(feat_flat, idx_flat):
    rows = idx_flat.shape[0]
    rows_pw = rows // NW
    nchunk = rows_pw // CHUNK
    kfn = functools.partial(
        pl.kernel,
        mesh=plsc.VectorSubcoreMesh(core_axis_name="c", subcore_axis_name="s"),
        out_type=jax.ShapeDtypeStruct((rows, D), jnp.float32),
        scratch_types=[
            pltpu.VMEM((rows_pw,), jnp.int32),
            pltpu.VMEM((CHUNK, D), jnp.float32),
            pltpu.VMEM((CHUNK, D), jnp.float32),
            pltpu.SemaphoreType.DMA,
            pltpu.SemaphoreType.DMA,
            pltpu.SemaphoreType.DMA,
            pltpu.SemaphoreType.DMA,
        ],
    )(functools.partial(_gather_body, rows_pw, nchunk))
    return kfn(feat_flat, idx_flat)


def _dense_body(gf_ref, feat_ref, ww1_ref, bw1_ref, ww2_ref, bw2_ref,
                wm1_ref, bm1_ref, gm1_ref, em1_ref,
                wm2_ref, bm2_ref, gm2_ref, em2_ref, out_ref):
    fq = feat_ref[0]
    gf = gf_ref[0]
    fq_rep = jnp.broadcast_to(fq[:, None, :], (RC, K, D)).reshape(RC * K, D)
    rel = gf - fq_rep

    ww1 = ww1_ref[...]
    bw1 = bw1_ref[...]
    ww2 = ww2_ref[...]
    bw2 = bw2_ref[...]

    h1 = jnp.maximum(jnp.dot(rel, ww1, preferred_element_type=jnp.float32)
                     + bw1, 0.0)
    h2 = jnp.dot(h1, ww2, preferred_element_type=jnp.float32) + bw2
    w = 1.0 / (1.0 + jnp.exp(-h2))

    accum = jnp.sum((rel * w).reshape(RC, K, D), axis=1)

    h1s = jnp.maximum(jnp.dot(fq, ww1, preferred_element_type=jnp.float32)
                      + bw1, 0.0)
    h2s = jnp.dot(h1s, ww2, preferred_element_type=jnp.float32) + bw2
    ws = 1.0 / (1.0 + jnp.exp(-h2s))

    res = fq + ws * fq - accum

    h = jnp.maximum(gm1_ref[...] * (jnp.dot(res, wm1_ref[...],
                                            preferred_element_type=jnp.float32)
                                    + bm1_ref[...]) + em1_ref[...], 0.0)
    out = gm2_ref[...] * (jnp.dot(h, wm2_ref[...],
                                  preferred_element_type=jnp.float32)
                          + bm2_ref[...]) + em2_ref[...]
    out_ref[0] = out


def _dense(gf, feat, ww1, bw1, ww2, bw2, wm1, bm1, gm1, em1,
           wm2, bm2, gm2, em2):
    bz = gf.shape[0]
    full = lambda r, c: pl.BlockSpec((r, c), lambda b, i: (0, 0))
    return pl.pallas_call(
        _dense_body,
        grid=(bz, N // RC),
        in_specs=[
            pl.BlockSpec((1, RC * K, D), lambda b, i: (b, i, 0)),
            pl.BlockSpec((1, RC, D), lambda b, i: (b, i, 0)),
            full(D, D), full(1, D), full(D, D), full(1, D),
            full(D, D), full(1, D), full(1, D), full(1, D),
            full(D, D), full(1, D), full(1, D), full(1, D),
        ],
        out_specs=pl.BlockSpec((1, RC, D), lambda b, i: (b, i, 0)),
        out_shape=jax.ShapeDtypeStruct((bz, N, D), jnp.float32),
    )(gf, feat, ww1, bw1, ww2, bw2, wm1, bm1, gm1, em1, wm2, bm2, gm2, em2)


def kernel(x, pos, W_mlp1, b_mlp1, g_mlp1, be_mlp1, W_mlp2, b_mlp2, g_mlp2,
           be_mlp2, W_w1, b_w1, W_w2, b_w2):
    feat = jnp.concatenate([x, pos], axis=-1)
    post = jnp.transpose(pos, (0, 2, 1))

    row = lambda v: v.reshape(1, D)
    wargs = (W_w1, row(b_w1), W_w2, row(b_w2),
             W_mlp1, row(b_mlp1), row(g_mlp1), row(be_mlp1),
             W_mlp2, row(b_mlp2), row(g_mlp2), row(be_mlp2))

    G = 4
    outs = []
    for b in range(0, BZ, G):
        nn_b = _knn(pos[b:b + G], post[b:b + G])
        gf_b = _sc_gather(feat[b:b + G].reshape(G * N, D),
                          nn_b.reshape(G * N * K))
        outs.append(_dense(gf_b.reshape(G, N * K, D), feat[b:b + G], *wargs))
    return jnp.concatenate(outs, axis=0)


# speedup vs baseline: 1.0816x; 1.0816x over previous
import functools

import jax
import jax.numpy as jnp
from jax import lax
from jax.experimental import pallas as pl
from jax.experimental.pallas import tpu as pltpu
from jax.experimental.pallas import tpu_sc as plsc

K = 16
N = 2048
BZ = 4
D = 128
RA = 256
RC = 128

NC = 2
NS = 16
NW = NC * NS
TOT_ROWS = BZ * N * K
ROWS_PER_W = TOT_ROWS // NW
CHUNK = 128
NCHUNK = ROWS_PER_W // CHUNK


def _knn_body(pos_ref, post_ref, nn_ref):
    b = pl.program_id(0)
    pq = pos_ref[0]
    pk = post_ref[0]
    dist = None
    for d in range(3):
        dd = pq[:, d:d + 1] - pk[d:d + 1, :]
        sq = dd * dd
        dist = sq if dist is None else dist + sq
    iota = lax.broadcasted_iota(jnp.int32, (RA, N), 1).astype(jnp.float32)
    kio = lax.broadcasted_iota(jnp.int32, (RA, K), 1)
    out = jnp.zeros((RA, K), jnp.float32)
    inf = jnp.float32(jnp.inf)
    for k in range(K):
        m = jnp.min(dist, axis=1, keepdims=True)
        cand = jnp.where(dist == m, iota, float(N))
        idx = jnp.min(cand, axis=1, keepdims=True)
        out = jnp.where(kio == k, idx, out)
        dist = jnp.where(cand == idx, inf, dist)
    nn_ref[0] = out.astype(jnp.int32) + b * N


def _knn(pos, post):
    bz = pos.shape[0]
    return pl.pallas_call(
        _knn_body,
        grid=(bz, N // RA),
        in_specs=[
            pl.BlockSpec((1, RA, 3), lambda b, i: (b, i, 0)),
            pl.BlockSpec((1, 3, N), lambda b, i: (b, 0, 0)),
        ],
        out_specs=pl.BlockSpec((1, RA, K), lambda b, i: (b, i, 0)),
        out_shape=jax.ShapeDtypeStruct((bz, N, K), jnp.int32),
    )(pos, post)


def _gather_body(rows_pw, nchunk, table_hbm, idx_hbm, out_hbm, idx_v,
                 rows0, rows1, g0, g1, s0, s1):
    NCHUNK = nchunk
    wid = lax.axis_index("s") * NC + lax.axis_index("c")
    wbase = pl.multiple_of(wid * rows_pw, rows_pw)
    pltpu.sync_copy(idx_hbm.at[pl.ds(wbase, rows_pw)], idx_v)

    rows = (rows0, rows1)
    gsem = (g0, g1)
    ssem = (s0, s1)

    def gather_start(i):
        buf = i % 2
        return pltpu.async_copy(
            table_hbm.at[idx_v.at[pl.ds(i * CHUNK, CHUNK)]],
            rows[buf], gsem[buf])

    def scatter_start(i):
        buf = i % 2
        base = pl.multiple_of(wbase + i * CHUNK, CHUNK)
        return pltpu.async_copy(rows[buf], out_hbm.at[pl.ds(base, CHUNK)],
                                ssem[buf])

    pending = {0: gather_start(0)}
    scat = {}
    for i in range(NCHUNK):
        buf = i % 2
        if i + 1 < NCHUNK:
            if i >= 1:
                scat[i - 1].wait()
            pending[i + 1] = gather_start(i + 1)
        pending[i].wait()
        scat[i] = scatter_start(i)
    scat[NCHUNK - 2].wait()
    scat[NCHUNK - 1].wait()


def _sc_gather(feat_flat, idx_flat):
    rows = idx_flat.shape[0]
    rows_pw = rows // NW
    nchunk = rows_pw // CHUNK
    kfn = functools.partial(
        pl.kernel,
        mesh=plsc.VectorSubcoreMesh(core_axis_name="c", subcore_axis_name="s"),
        out_type=jax.ShapeDtypeStruct((rows, D), jnp.float32),
        scratch_types=[
            pltpu.VMEM((rows_pw,), jnp.int32),
            pltpu.VMEM((CHUNK, D), jnp.float32),
            pltpu.VMEM((CHUNK, D), jnp.float32),
            pltpu.SemaphoreType.DMA,
            pltpu.SemaphoreType.DMA,
            pltpu.SemaphoreType.DMA,
            pltpu.SemaphoreType.DMA,
        ],
    )(functools.partial(_gather_body, rows_pw, nchunk))
    return kfn(feat_flat, idx_flat)


def _dense_body(gf_ref, feat_ref, ww1_ref, bw1_ref, ww2_ref, bw2_ref,
                wm1_ref, bm1_ref, gm1_ref, em1_ref,
                wm2_ref, bm2_ref, gm2_ref, em2_ref, out_ref):
    fq = feat_ref[0]
    gf = gf_ref[0]
    fq_rep = jnp.broadcast_to(fq[:, None, :], (RC, K, D)).reshape(RC * K, D)
    rel = gf - fq_rep

    ww1 = ww1_ref[...]
    bw1 = bw1_ref[...]
    ww2 = ww2_ref[...]
    bw2 = bw2_ref[...]

    h1 = jnp.maximum(jnp.dot(rel, ww1, preferred_element_type=jnp.float32)
                     + bw1, 0.0)
    h2 = jnp.dot(h1, ww2, preferred_element_type=jnp.float32) + bw2
    w = 1.0 / (1.0 + jnp.exp(-h2))

    accum = jnp.sum((rel * w).reshape(RC, K, D), axis=1)

    h1s = jnp.maximum(jnp.dot(fq, ww1, preferred_element_type=jnp.float32)
                      + bw1, 0.0)
    h2s = jnp.dot(h1s, ww2, preferred_element_type=jnp.float32) + bw2
    ws = 1.0 / (1.0 + jnp.exp(-h2s))

    res = fq + ws * fq - accum

    h = jnp.maximum(gm1_ref[...] * (jnp.dot(res, wm1_ref[...],
                                            preferred_element_type=jnp.float32)
                                    + bm1_ref[...]) + em1_ref[...], 0.0)
    out = gm2_ref[...] * (jnp.dot(h, wm2_ref[...],
                                  preferred_element_type=jnp.float32)
                          + bm2_ref[...]) + em2_ref[...]
    out_ref[0] = out


def _dense(gf, feat, ww1, bw1, ww2, bw2, wm1, bm1, gm1, em1,
           wm2, bm2, gm2, em2):
    bz = gf.shape[0]
    full = lambda r, c: pl.BlockSpec((r, c), lambda b, i: (0, 0))
    return pl.pallas_call(
        _dense_body,
        grid=(bz, N // RC),
        in_specs=[
            pl.BlockSpec((1, RC * K, D), lambda b, i: (b, i, 0)),
            pl.BlockSpec((1, RC, D), lambda b, i: (b, i, 0)),
            full(D, D), full(1, D), full(D, D), full(1, D),
            full(D, D), full(1, D), full(1, D), full(1, D),
            full(D, D), full(1, D), full(1, D), full(1, D),
        ],
        out_specs=pl.BlockSpec((1, RC, D), lambda b, i: (b, i, 0)),
        out_shape=jax.ShapeDtypeStruct((bz, N, D), jnp.float32),
    )(gf, feat, ww1, bw1, ww2, bw2, wm1, bm1, gm1, em1, wm2, bm2, gm2, em2)


def kernel(x, pos, W_mlp1, b_mlp1, g_mlp1, be_mlp1, W_mlp2, b_mlp2, g_mlp2,
           be_mlp2, W_w1, b_w1, W_w2, b_w2):
    feat = jnp.concatenate([x, pos], axis=-1)
    post = jnp.transpose(pos, (0, 2, 1))

    row = lambda v: v.reshape(1, D)
    wargs = (W_w1, row(b_w1), W_w2, row(b_w2),
             W_mlp1, row(b_mlp1), row(g_mlp1), row(be_mlp1),
             W_mlp2, row(b_mlp2), row(g_mlp2), row(be_mlp2))

    G = 2
    outs = []
    for b in range(0, BZ, G):
        nn_b = _knn(pos[b:b + G], post[b:b + G])
        gf_b = _sc_gather(feat[b:b + G].reshape(G * N, D),
                          nn_b.reshape(G * N * K))
        outs.append(_dense(gf_b.reshape(G, N * K, D), feat[b:b + G], *wargs))
    return jnp.concatenate(outs, axis=0)
